# Initial kernel scaffold; baseline (speedup 1.0000x reference)
import jax
import jax.numpy as jnp
from jax.experimental import pallas as pl


def kernel(inputs, targets, features):
    raise NotImplementedError("write your pallas kernel here")


# speedup vs baseline: 5.5922x; 5.5922x over previous
import functools

import jax
import jax.numpy as jnp
from jax import lax
from jax.experimental import pallas as pl
from jax.experimental.pallas import tpu as pltpu
from jax.experimental.pallas import tpu_sc as plsc

TEMP = 0.05


def _sc_gather_rows(features, targets):
    n_rows, d = features.shape
    b = targets.shape[0]
    try:
        info = plsc.get_sparse_core_info()
        nc, ns = info.num_cores, info.num_subcores
    except Exception:
        nc, ns = 2, 16
    nw = nc * ns
    b_per_w = b // nw
    mesh = plsc.VectorSubcoreMesh(core_axis_name="c", subcore_axis_name="s")

    @functools.partial(
        pl.kernel,
        mesh=mesh,
        out_type=jax.ShapeDtypeStruct((b, d), jnp.float32),
        scratch_types=[
            pltpu.VMEM((b_per_w,), jnp.int32),
            pltpu.VMEM((b_per_w, d), jnp.float32),
            pltpu.SemaphoreType.DMA,
        ],
    )
    def gather_kernel(table_hbm, idx_hbm, out_hbm, idx_v, rows_v, sem):
        wid = lax.axis_index("s") * nc + lax.axis_index("c")
        base = wid * b_per_w
        pltpu.sync_copy(idx_hbm.at[pl.ds(base, b_per_w)], idx_v)
        pltpu.async_copy(table_hbm.at[idx_v], rows_v, sem).wait()
        pltpu.sync_copy(rows_v, out_hbm.at[pl.ds(base, b_per_w)])

    return gather_kernel(features, targets)


def _tc_loss(inputs, features, tgt_rows):
    b, d = inputs.shape
    n = features.shape[0]
    chunk = 2000
    grid = n // chunk

    def body(x_ref, f_ref, t_ref, out_ref, xs_ref, acc_ref):
        i = pl.program_id(0)

        @pl.when(i == 0)
        def _prologue():
            x = x_ref[...]
            nrm = jnp.sum(x * x, axis=1, keepdims=True)
            xs_ref[...] = x * (1.0 / (jnp.sqrt(nrm) * TEMP))
            acc_ref[...] = jnp.zeros_like(acc_ref)

        s = lax.dot_general(
            xs_ref[...], f_ref[...], (((1,), (1,)), ((), ())),
            preferred_element_type=jnp.float32,
        )
        acc_ref[...] += jnp.sum(jnp.exp(s), axis=1, keepdims=True)

        @pl.when(i == grid - 1)
        def _epilogue():
            tgt = jnp.sum(xs_ref[...] * t_ref[...], axis=1, keepdims=True)
            nll = jnp.log(acc_ref[...]) - tgt
            out_ref[0, 0] = jnp.mean(nll)

    out = pl.pallas_call(
        body,
        grid=(grid,),
        in_specs=[
            pl.BlockSpec((b, d), lambda i: (0, 0)),
            pl.BlockSpec((chunk, d), lambda i: (i, 0)),
            pl.BlockSpec((b, d), lambda i: (0, 0)),
        ],
        out_specs=pl.BlockSpec((1, 1), lambda i: (0, 0), memory_space=pltpu.SMEM),
        out_shape=jax.ShapeDtypeStruct((1, 1), jnp.float32),
        scratch_shapes=[
            pltpu.VMEM((b, d), jnp.float32),
            pltpu.VMEM((b, 1), jnp.float32),
        ],
    )(inputs, features, tgt_rows)
    return out[0, 0]


def kernel(inputs, targets, features):
    tgt_rows = _sc_gather_rows(features, targets)
    return _tc_loss(inputs, features, tgt_rows)
